# baseline (speedup 1.0000x reference)
import functools

import jax
import jax.numpy as jnp
from jax import lax
from jax.experimental import pallas as pl
from jax.experimental.pallas import tpu as pltpu
from jax.experimental.pallas import tpu_sc as plsc

B, P, N, G = 16, 16, 16384, 64
G3 = G * G * G
LANES = 16
NW = 32
PAIRS_PER_W = (B * P) // NW
CHUNK = 2048
NCH = N // CHUNK
NT = PAIRS_PER_W * NCH
PK = 2048
WCELLS = (B * G3) // NW


def _floor_f32(x):
    t = x.astype(jnp.int32)
    tf = t.astype(jnp.float32)
    return jnp.where(tf > x, t - 1, t)


def _pack_body(cpx_hbm, cpy_hbm, cpz_hbm, vox_hbm, packed_hbm,
               sx_v, sy_v, sz_v, sv_v, pk_v):
    wid = lax.axis_index("c") * 16 + lax.axis_index("s")
    cells0 = wid * WCELLS
    lane4 = lax.iota(jnp.int32, LANES) * 4

    def pack_tile(t, carry):
        base = cells0 + t * PK
        pltpu.sync_copy(cpx_hbm.at[pl.ds(base, PK)], sx_v)
        pltpu.sync_copy(cpy_hbm.at[pl.ds(base, PK)], sy_v)
        pltpu.sync_copy(cpz_hbm.at[pl.ds(base, PK)], sz_v)
        pltpu.sync_copy(vox_hbm.at[pl.ds(base, PK)], sv_v)

        def step(i, carry):
            lo = i * LANES
            tpos = lo * 4 + lane4
            plsc.store_scatter(pk_v, [tpos], sx_v[pl.ds(lo, LANES)])
            plsc.store_scatter(pk_v, [tpos + 1], sy_v[pl.ds(lo, LANES)])
            plsc.store_scatter(pk_v, [tpos + 2], sz_v[pl.ds(lo, LANES)])
            plsc.store_scatter(pk_v, [tpos + 3], sv_v[pl.ds(lo, LANES)])
            return carry

        lax.fori_loop(0, PK // LANES, step, 0)
        pltpu.sync_copy(pk_v, packed_hbm.at[pl.ds(base * 4, PK * 4)])
        return carry

    lax.fori_loop(0, WCELLS // PK, pack_tile, 0)


_sc_pack = functools.partial(
    pl.kernel,
    out_type=jax.ShapeDtypeStruct((B * G3 * 4,), jnp.float32),
    mesh=plsc.VectorSubcoreMesh(core_axis_name="c", subcore_axis_name="s"),
    scratch_types=[
        pltpu.VMEM((PK,), jnp.float32),
        pltpu.VMEM((PK,), jnp.float32),
        pltpu.VMEM((PK,), jnp.float32),
        pltpu.VMEM((PK,), jnp.float32),
        pltpu.VMEM((PK * 4,), jnp.float32),
    ],
    compiler_params=pltpu.CompilerParams(
        needs_layout_passes=False, use_tc_tiling_on_sc=False
    ),
)(_pack_body)


def _sc_body(pts_hbm, par_hbm, packed_hbm, out_hbm,
             pts_v, par_v, acc_v,
             ix0, iy0, iz0, iv0, rx0, ry0, rz0, gx0, gy0, gz0, gv0, sem0,
             ix1, iy1, iz1, iv1, rx1, ry1, rz1, gx1, gy1, gz1, gv1, sem1):
    cid = lax.axis_index("c")
    sid = lax.axis_index("s")
    wid = cid * 16 + sid
    b = wid // 2
    p0 = (wid % 2) * PAIRS_PER_W

    pltpu.sync_copy(pts_hbm.at[pl.ds(b * 3 * N, 3 * N)], pts_v)
    pltpu.sync_copy(
        par_hbm.at[pl.ds((b * P + p0) * 4 * LANES, PAIRS_PER_W * 4 * LANES)],
        par_v)

    bufs = (
        (ix0, iy0, iz0, iv0, rx0, ry0, rz0, gx0, gy0, gz0, gv0, sem0),
        (ix1, iy1, iz1, iv1, rx1, ry1, rz1, gx1, gy1, gz1, gv1, sem1),
    )

    def pass1(t, buf):
        ix_b, iy_b, iz_b, iv_b = buf[0], buf[1], buf[2], buf[3]
        rx_v, ry_v, rz_v = buf[4], buf[5], buf[6]
        j = t // NCH
        cbase = (t % NCH) * CHUNK
        po = j * 4 * LANES
        nx = par_v[pl.ds(po, LANES)]
        ny = par_v[pl.ds(po + LANES, LANES)]
        nz = par_v[pl.ds(po + 2 * LANES, LANES)]
        dd = par_v[pl.ds(po + 3 * LANES, LANES)]

        def step(i, carry):
            o = cbase + i * LANES
            px = pts_v[pl.ds(o, LANES)]
            py = pts_v[pl.ds(N + o, LANES)]
            pz = pts_v[pl.ds(2 * N + o, LANES)]
            inner = px * nx + py * ny + pz * nz + dd
            t2 = inner + inner
            rx = px - t2 * nx
            ry = py - t2 * ny
            rz = pz - t2 * nz
            ix = _floor_f32((rx + 0.5) * float(G))
            iy = _floor_f32((ry + 0.5) * float(G))
            iz = _floor_f32((rz + 0.5) * float(G))
            cell4 = (jnp.clip(ix * (G * G) + iy * G + iz, 0, G3 - 1)
                     + b * G3) * 4
            lo = i * LANES
            ix_b[pl.ds(lo, LANES)] = cell4
            iy_b[pl.ds(lo, LANES)] = cell4 + 1
            iz_b[pl.ds(lo, LANES)] = cell4 + 2
            iv_b[pl.ds(lo, LANES)] = cell4 + 3
            rx_v[pl.ds(lo, LANES)] = rx
            ry_v[pl.ds(lo, LANES)] = ry
            rz_v[pl.ds(lo, LANES)] = rz
            return carry

        lax.fori_loop(0, CHUNK // LANES, step, 0)

    def descs(buf):
        sem = buf[11]
        return [
            pltpu.make_async_copy(packed_hbm.at[idx_b], dst, sem)
            for idx_b, dst in ((buf[0], buf[7]), (buf[1], buf[8]),
                               (buf[2], buf[9]), (buf[3], buf[10]))
        ]

    def start_dma(buf):
        for d in descs(buf):
            d.start()

    def wait_dma(buf):
        for d in descs(buf):
            d.wait()

    def pass2(buf, acc):
        rx_v, ry_v, rz_v = buf[4], buf[5], buf[6]
        gx_v, gy_v, gz_v, gv_v = buf[7], buf[8], buf[9], buf[10]

        def step(i, acc):
            ro = i * LANES
            dx = rx_v[pl.ds(ro, LANES)] - gx_v[pl.ds(ro, LANES)]
            dy = ry_v[pl.ds(ro, LANES)] - gy_v[pl.ds(ro, LANES)]
            dz = rz_v[pl.ds(ro, LANES)] - gz_v[pl.ds(ro, LANES)]
            m = 1.0 - gv_v[pl.ds(ro, LANES)]
            return acc + (dx * dx + dy * dy + dz * dz) * (m * m)

        return lax.fori_loop(0, CHUNK // LANES, step, acc)

    pass1(0, bufs[0])
    start_dma(bufs[0])

    def body(g, acc):
        pass1(2 * g + 1, bufs[1])
        start_dma(bufs[1])
        wait_dma(bufs[0])
        acc = pass2(bufs[0], acc)
        pass1(2 * g + 2, bufs[0])
        start_dma(bufs[0])
        wait_dma(bufs[1])
        return pass2(bufs[1], acc)

    acc = lax.fori_loop(0, NT // 2 - 1, body, jnp.zeros((LANES,), jnp.float32))
    pass1(NT - 1, bufs[1])
    start_dma(bufs[1])
    wait_dma(bufs[0])
    acc = pass2(bufs[0], acc)
    wait_dma(bufs[1])
    acc = pass2(bufs[1], acc)

    acc_v[...] = acc
    pltpu.sync_copy(acc_v, out_hbm.at[pl.ds(wid * LANES, LANES)])


def _buf_types():
    return [
        pltpu.VMEM((CHUNK,), jnp.int32),
        pltpu.VMEM((CHUNK,), jnp.int32),
        pltpu.VMEM((CHUNK,), jnp.int32),
        pltpu.VMEM((CHUNK,), jnp.int32),
        pltpu.VMEM((CHUNK,), jnp.float32),
        pltpu.VMEM((CHUNK,), jnp.float32),
        pltpu.VMEM((CHUNK,), jnp.float32),
        pltpu.VMEM((CHUNK,), jnp.float32),
        pltpu.VMEM((CHUNK,), jnp.float32),
        pltpu.VMEM((CHUNK,), jnp.float32),
        pltpu.VMEM((CHUNK,), jnp.float32),
        pltpu.SemaphoreType.DMA,
    ]


_sc_loss = functools.partial(
    pl.kernel,
    out_type=jax.ShapeDtypeStruct((NW * LANES,), jnp.float32),
    mesh=plsc.VectorSubcoreMesh(core_axis_name="c", subcore_axis_name="s"),
    scratch_types=[
        pltpu.VMEM((3 * N,), jnp.float32),
        pltpu.VMEM((PAIRS_PER_W * 4 * LANES,), jnp.float32),
        pltpu.VMEM((LANES,), jnp.float32),
    ] + _buf_types() + _buf_types(),
    compiler_params=pltpu.CompilerParams(use_tc_tiling_on_sc=False),
)(_sc_body)


def kernel(points, closest_points, voxel, planes):
    eps = 1e-12
    ns = planes[..., :3]
    ds = planes[..., 3]
    ns_norm = jnp.sqrt(jnp.sum(ns * ns, axis=2, keepdims=True))
    n_unit = ns / (ns_norm + eps)
    d_unit = ds[..., None] / (ns_norm + eps)
    params = jnp.concatenate([n_unit, d_unit], axis=-1)
    par_1d = jnp.broadcast_to(params[..., None], (B, P, 4, LANES)).reshape(-1)
    pts_1d = jnp.transpose(points, (0, 2, 1)).reshape(-1)
    cpx = closest_points[..., 0].reshape(-1)
    cpy = closest_points[..., 1].reshape(-1)
    cpz = closest_points[..., 2].reshape(-1)
    vox_1d = voxel.reshape(-1)
    packed = _sc_pack(cpx, cpy, cpz, vox_1d)
    partial = _sc_loss(pts_1d, par_1d, packed)
    return jnp.sum(partial) / (B * P)


# speedup vs baseline: 1.5055x; 1.5055x over previous
import functools

import jax
import jax.numpy as jnp
from jax import lax
from jax.experimental import pallas as pl
from jax.experimental.pallas import tpu as pltpu
from jax.experimental.pallas import tpu_sc as plsc

B, P, N, G = 16, 16, 16384, 64
G3 = G * G * G
LANES = 16
NW = 32
PAIRS_PER_W = (B * P) // NW
CHUNK = 2048
NCH = N // CHUNK
NT = PAIRS_PER_W * NCH


def _floor_f32(x):
    t = x.astype(jnp.int32)
    tf = t.astype(jnp.float32)
    return jnp.where(tf > x, t - 1, t)


def _sc_body(pts_hbm, par_hbm, cpx_hbm, cpy_hbm, cpz_hbm, vox_hbm, out_hbm,
             pts_v, par_v, acc_v,
             idx0, rx0, ry0, rz0, gx0, gy0, gz0, gv0, sem0,
             idx1, rx1, ry1, rz1, gx1, gy1, gz1, gv1, sem1,
             idx2, rx2, ry2, rz2, gx2, gy2, gz2, gv2, sem2,
             idx3, rx3, ry3, rz3, gx3, gy3, gz3, gv3, sem3):
    cid = lax.axis_index("c")
    sid = lax.axis_index("s")
    wid = cid * 16 + sid
    b = wid // 2
    p0 = (wid % 2) * PAIRS_PER_W

    pltpu.sync_copy(pts_hbm.at[pl.ds(b * 3 * N, 3 * N)], pts_v)
    pltpu.sync_copy(
        par_hbm.at[pl.ds((b * P + p0) * 4 * LANES, PAIRS_PER_W * 4 * LANES)],
        par_v)

    bufs = (
        (idx0, rx0, ry0, rz0, gx0, gy0, gz0, gv0, sem0),
        (idx1, rx1, ry1, rz1, gx1, gy1, gz1, gv1, sem1),
        (idx2, rx2, ry2, rz2, gx2, gy2, gz2, gv2, sem2),
        (idx3, rx3, ry3, rz3, gx3, gy3, gz3, gv3, sem3),
    )

    def pass1(t, buf):
        idx_b, rx_v, ry_v, rz_v = buf[0], buf[1], buf[2], buf[3]
        j = t // NCH
        cbase = (t % NCH) * CHUNK
        po = j * 4 * LANES
        nx = par_v[pl.ds(po, LANES)]
        ny = par_v[pl.ds(po + LANES, LANES)]
        nz = par_v[pl.ds(po + 2 * LANES, LANES)]
        dd = par_v[pl.ds(po + 3 * LANES, LANES)]

        def step(i, carry):
            o = cbase + i * LANES
            px = pts_v[pl.ds(o, LANES)]
            py = pts_v[pl.ds(N + o, LANES)]
            pz = pts_v[pl.ds(2 * N + o, LANES)]
            inner = px * nx + py * ny + pz * nz + dd
            t2 = inner + inner
            rx = px - t2 * nx
            ry = py - t2 * ny
            rz = pz - t2 * nz
            ix = _floor_f32((rx + 0.5) * float(G))
            iy = _floor_f32((ry + 0.5) * float(G))
            iz = _floor_f32((rz + 0.5) * float(G))
            cell = jnp.clip(ix * (G * G) + iy * G + iz, 0, G3 - 1)
            lo = i * LANES
            idx_b[pl.ds(lo, LANES)] = cell + b * G3
            rx_v[pl.ds(lo, LANES)] = rx
            ry_v[pl.ds(lo, LANES)] = ry
            rz_v[pl.ds(lo, LANES)] = rz
            return carry

        lax.fori_loop(0, CHUNK // LANES, step, 0)

    def descs(buf):
        idx_b = buf[0]
        sem = buf[8]
        return [
            pltpu.make_async_copy(tab.at[idx_b], dst, sem)
            for tab, dst in ((cpx_hbm, buf[4]), (cpy_hbm, buf[5]),
                             (cpz_hbm, buf[6]), (vox_hbm, buf[7]))
        ]

    def start_dma(buf):
        for d in descs(buf):
            d.start()

    def wait_dma(buf):
        for d in descs(buf):
            d.wait()

    def pass2(buf, acc):
        rx_v, ry_v, rz_v = buf[1], buf[2], buf[3]
        gx_v, gy_v, gz_v, gv_v = buf[4], buf[5], buf[6], buf[7]

        def step(i, acc):
            ro = i * LANES
            dx = rx_v[pl.ds(ro, LANES)] - gx_v[pl.ds(ro, LANES)]
            dy = ry_v[pl.ds(ro, LANES)] - gy_v[pl.ds(ro, LANES)]
            dz = rz_v[pl.ds(ro, LANES)] - gz_v[pl.ds(ro, LANES)]
            m = 1.0 - gv_v[pl.ds(ro, LANES)]
            return acc + (dx * dx + dy * dy + dz * dz) * (m * m)

        return lax.fori_loop(0, CHUNK // LANES, step, acc)

    for k in range(3):
        pass1(k, bufs[k])
        start_dma(bufs[k])

    def body(g, acc):
        for k in range(4):
            t = 4 * g + k

            @pl.when(t + 3 < NT)
            def _():
                pass1(t + 3, bufs[(k + 3) % 4])
                start_dma(bufs[(k + 3) % 4])

            wait_dma(bufs[k])
            acc = pass2(bufs[k], acc)
        return acc

    acc = lax.fori_loop(0, NT // 4, body, jnp.zeros((LANES,), jnp.float32))

    acc_v[...] = acc
    pltpu.sync_copy(acc_v, out_hbm.at[pl.ds(wid * LANES, LANES)])


def _buf_types():
    return [
        pltpu.VMEM((CHUNK,), jnp.int32),
        pltpu.VMEM((CHUNK,), jnp.float32),
        pltpu.VMEM((CHUNK,), jnp.float32),
        pltpu.VMEM((CHUNK,), jnp.float32),
        pltpu.VMEM((CHUNK,), jnp.float32),
        pltpu.VMEM((CHUNK,), jnp.float32),
        pltpu.VMEM((CHUNK,), jnp.float32),
        pltpu.VMEM((CHUNK,), jnp.float32),
        pltpu.SemaphoreType.DMA,
    ]


_sc_loss = functools.partial(
    pl.kernel,
    out_type=jax.ShapeDtypeStruct((NW * LANES,), jnp.float32),
    mesh=plsc.VectorSubcoreMesh(core_axis_name="c", subcore_axis_name="s"),
    scratch_types=[
        pltpu.VMEM((3 * N,), jnp.float32),
        pltpu.VMEM((PAIRS_PER_W * 4 * LANES,), jnp.float32),
        pltpu.VMEM((LANES,), jnp.float32),
    ] + _buf_types() + _buf_types() + _buf_types() + _buf_types(),
    compiler_params=pltpu.CompilerParams(use_tc_tiling_on_sc=False),
)(_sc_body)


def kernel(points, closest_points, voxel, planes):
    eps = 1e-12
    ns = planes[..., :3]
    ds = planes[..., 3]
    ns_norm = jnp.sqrt(jnp.sum(ns * ns, axis=2, keepdims=True))
    n_unit = ns / (ns_norm + eps)
    d_unit = ds[..., None] / (ns_norm + eps)
    params = jnp.concatenate([n_unit, d_unit], axis=-1)
    par_1d = jnp.broadcast_to(params[..., None], (B, P, 4, LANES)).reshape(-1)
    pts_1d = jnp.transpose(points, (0, 2, 1)).reshape(-1)
    cpx = closest_points[..., 0].reshape(-1)
    cpy = closest_points[..., 1].reshape(-1)
    cpz = closest_points[..., 2].reshape(-1)
    vox_1d = voxel.reshape(-1)
    partial = _sc_loss(pts_1d, par_1d, cpx, cpy, cpz, vox_1d)
    return jnp.sum(partial) / (B * P)
